# baseline (speedup 1.0000x reference)
import functools

import jax
import jax.numpy as jnp
from jax import lax
from jax.experimental import pallas as pl
from jax.experimental.pallas import tpu as pltpu
from jax.experimental.pallas import tpu_sc as plsc

N_NODES = 10000
N_EDGES = 320000
F_IN = 128
HID = 16
N_CLASSES = 40

NC, NS = 2, 16
NW = NC * NS
N_PAD = 10240
ROWS_T = N_PAD // NS
ROWS_W = N_PAD // NW
E_W = N_EDGES // NW
CH = 120
NCH = E_W // CH
TAIL = E_W - NCH * CH
C_PAD = 48

_mesh = plsc.VectorSubcoreMesh(core_axis_name="c", subcore_axis_name="s",
                               num_cores=NC, num_subcores=NS)
_sc_params = pltpu.CompilerParams(use_tc_tiling_on_sc=False)


def _rsqrt16(d):
    x = jnp.maximum(d, 1.0)
    i = lax.bitcast_convert_type(x, jnp.int32)
    i = jnp.int32(0x5F3759DF) - lax.shift_right_arithmetic(i, 1)
    y = lax.bitcast_convert_type(i, jnp.float32)
    for _ in range(4):
        y = y * (1.5 - 0.5 * x * y * y)
    return y


_KI = 16
_NBUF, _DG, _DS = 12, 6, 6
_LEAD = 2


def _idx_ring(edge_hbm, ebase, si, di, isem):
    def start_l(i):
        off = ebase + i * CH
        pltpu.async_copy(edge_hbm.at[0, pl.ds(off, CH)],
                         si.at[lax.rem(i, _KI)], isem)
        pltpu.async_copy(edge_hbm.at[1, pl.ds(off, CH)],
                         di.at[lax.rem(i, _KI)], isem)

    def wait_l():
        pltpu.make_async_copy(edge_hbm.at[0, pl.ds(0, CH)],
                              si.at[0], isem).wait()
        pltpu.make_async_copy(edge_hbm.at[0, pl.ds(0, CH)],
                              di.at[0], isem).wait()

    return start_l, wait_l


def _agg_pipeline(edge_hbm, ebase, gather_src_at,
                  si, di, ti_v, rows_v, tail_v, acc, isem, gsem, ssem):
    start_l, wait_l = _idx_ring(edge_hbm, ebase, si, di, isem)

    def start_g(i):
        pltpu.async_copy(gather_src_at(si.at[lax.rem(i, _KI)]),
                         rows_v.at[lax.rem(i, _NBUF)], gsem)

    def wait_g():
        pltpu.make_async_copy(gather_src_at(si.at[0]),
                              rows_v.at[0], gsem).wait()

    def start_s(i):
        pltpu.async_copy(rows_v.at[lax.rem(i, _NBUF)],
                         acc.at[di.at[lax.rem(i, _KI)]], ssem, add=True)

    def wait_s():
        pltpu.make_async_copy(rows_v.at[0],
                              acc.at[di.at[0]], ssem).wait()

    for i in range(_DG + _LEAD):
        start_l(i)
    for i in range(_DG):
        wait_l()
        start_g(i)

    def body(i, carry):
        wait_g()
        pl.when(i >= _DS)(wait_s)
        pl.when(i + _DG + _LEAD < NCH)(lambda: start_l(i + _DG + _LEAD))

        def advance():
            wait_l()
            start_g(i + _DG)

        pl.when(i + _DG < NCH)(advance)
        start_s(i)
        return carry

    lax.fori_loop(0, NCH, body, 0)
    for _ in range(_DS):
        wait_s()

    toff = ebase + NCH * CH
    pltpu.sync_copy(edge_hbm.at[0, pl.ds(toff, TAIL)], ti_v.at[0])
    pltpu.sync_copy(edge_hbm.at[1, pl.ds(toff, TAIL)], ti_v.at[1])
    pltpu.async_copy(gather_src_at(ti_v.at[0]), tail_v, gsem).wait()
    pltpu.sync_copy(tail_v, acc.at[ti_v.at[1]], add=True)


@functools.partial(
    pl.kernel,
    out_type=(
        jax.ShapeDtypeStruct((NC, N_PAD), jnp.float32),
        jax.ShapeDtypeStruct((NC, N_PAD), jnp.float32),
    ),
    mesh=_mesh,
    scratch_types=[
        pltpu.VMEM((_KI, CH), jnp.int32),
        pltpu.VMEM((_KI, CH), jnp.int32),
        pltpu.VMEM((2, TAIL), jnp.int32),
        pltpu.VMEM((CH,), jnp.float32),
        pltpu.VMEM((TAIL,), jnp.float32),
        pltpu.VMEM_SHARED((N_PAD,), jnp.float32),
        pltpu.VMEM_SHARED((N_PAD,), jnp.float32),
        pltpu.SemaphoreType.DMA,
        pltpu.SemaphoreType.DMA,
    ],
    compiler_params=_sc_params,
)
def _sc_degrees(edge_hbm, ones_hbm, zeros_hbm,
                degout_hbm, degin_hbm,
                si, di, ti_v, ones_v, ones_t, acc_o, acc_i, isem, ssem):
    c = lax.axis_index("c")
    s = lax.axis_index("s")
    w = c * NS + s
    ebase = w * E_W
    sl = pl.ds(s * ROWS_T, ROWS_T)
    pltpu.sync_copy(zeros_hbm.at[sl], acc_o.at[sl])
    pltpu.sync_copy(zeros_hbm.at[sl], acc_i.at[sl])
    pltpu.sync_copy(ones_hbm, ones_v)
    pltpu.sync_copy(ones_hbm.at[pl.ds(0, TAIL)], ones_t)
    plsc.subcore_barrier()

    start_l, wait_l = _idx_ring(edge_hbm, ebase, si, di, isem)
    depth = 8

    def wait_s():
        pltpu.make_async_copy(ones_v, acc_o.at[si.at[0]], ssem).wait()

    def wait_s2():
        wait_s()
        wait_s()

    for i in range(_LEAD):
        start_l(i)

    def body(i, carry):
        pl.when(i + _LEAD < NCH)(lambda: start_l(i + _LEAD))
        wait_l()
        pl.when(i >= depth)(wait_s2)
        m = lax.rem(i, _KI)
        pltpu.async_copy(ones_v, acc_o.at[si.at[m]], ssem, add=True)
        pltpu.async_copy(ones_v, acc_i.at[di.at[m]], ssem, add=True)
        return carry

    lax.fori_loop(0, NCH, body, 0)
    for _ in range(2 * depth):
        wait_s()

    toff = ebase + NCH * CH
    pltpu.sync_copy(edge_hbm.at[0, pl.ds(toff, TAIL)], ti_v.at[0])
    pltpu.sync_copy(edge_hbm.at[1, pl.ds(toff, TAIL)], ti_v.at[1])
    pltpu.sync_copy(ones_t, acc_o.at[ti_v.at[0]], add=True)
    pltpu.sync_copy(ones_t, acc_i.at[ti_v.at[1]], add=True)

    plsc.subcore_barrier()
    pltpu.sync_copy(acc_o.at[sl], degout_hbm.at[c, sl])
    pltpu.sync_copy(acc_i.at[sl], degin_hbm.at[c, sl])


@functools.partial(
    pl.kernel,
    out_type=jax.ShapeDtypeStruct((NC, N_PAD, HID), jnp.float32),
    mesh=_mesh,
    scratch_types=[
        pltpu.VMEM((_KI, CH), jnp.int32),
        pltpu.VMEM((_KI, CH), jnp.int32),
        pltpu.VMEM((2, TAIL), jnp.int32),
        pltpu.VMEM((_NBUF, CH, HID), jnp.float32),
        pltpu.VMEM((TAIL, HID), jnp.float32),
        pltpu.VMEM((ROWS_T, HID), jnp.float32),
        pltpu.VMEM((ROWS_T,), jnp.float32),
        pltpu.VMEM((ROWS_T,), jnp.float32),
        pltpu.VMEM((ROWS_T,), jnp.float32),
        pltpu.VMEM_SHARED((N_PAD, HID), jnp.float32),
        pltpu.VMEM_SHARED((N_PAD, HID), jnp.float32),
        pltpu.SemaphoreType.DMA,
        pltpu.SemaphoreType.DMA,
        pltpu.SemaphoreType.DMA,
    ],
    compiler_params=_sc_params,
)
def _sc_agg16(xwun_hbm, degs_hbm, edge_hbm, zeros_hbm, out_hbm,
              si, di, ti_v, rows_v, tail_v, xrows_v, dg0_v, dg1_v, norm_v,
              table, acc, isem, gsem, ssem):
    c = lax.axis_index("c")
    s = lax.axis_index("s")
    w = c * NS + s
    sl = pl.ds(s * ROWS_T, ROWS_T)

    pltpu.sync_copy(xwun_hbm.at[sl], xrows_v)
    pltpu.sync_copy(degs_hbm.at[0, sl], dg0_v)
    pltpu.sync_copy(degs_hbm.at[1, sl], dg1_v)
    pltpu.sync_copy(zeros_hbm.at[sl], acc.at[sl])

    def nbody(j, carry):
        d = dg0_v[pl.ds(j * 16, 16)] + dg1_v[pl.ds(j * 16, 16)]
        norm_v[pl.ds(j * 16, 16)] = _rsqrt16(d)
        return carry

    lax.fori_loop(0, ROWS_T // 16, nbody, 0)

    def sbody(j, carry):
        nv = norm_v[pl.ds(j * 16, 16)]
        for t in range(16):
            i = j * 16 + t
            xrows_v[i, :] = xrows_v[i, :] * nv[t]
        return carry

    lax.fori_loop(0, ROWS_T // 16, sbody, 0)
    pltpu.sync_copy(xrows_v, table.at[sl])
    plsc.subcore_barrier()

    _agg_pipeline(edge_hbm, w * E_W, lambda idx: table.at[idx],
                  si, di, ti_v, rows_v, tail_v, acc, isem, gsem, ssem)
    plsc.subcore_barrier()
    pltpu.sync_copy(acc.at[sl], out_hbm.at[c, sl])


@functools.partial(
    pl.kernel,
    out_type=jax.ShapeDtypeStruct((NC, N_PAD, C_PAD), jnp.float32),
    mesh=_mesh,
    scratch_types=[
        pltpu.VMEM((_KI, CH), jnp.int32),
        pltpu.VMEM((_KI, CH), jnp.int32),
        pltpu.VMEM((2, TAIL), jnp.int32),
        pltpu.VMEM((_NBUF, CH, C_PAD), jnp.float32),
        pltpu.VMEM((TAIL, C_PAD), jnp.float32),
        pltpu.VMEM_SHARED((N_PAD, C_PAD), jnp.float32),
        pltpu.SemaphoreType.DMA,
        pltpu.SemaphoreType.DMA,
        pltpu.SemaphoreType.DMA,
    ],
    compiler_params=_sc_params,
)
def _sc_agg48(rows_hbm, edge_hbm, zeros_hbm, out_hbm,
              si, di, ti_v, rows_v, tail_v, acc, isem, gsem, ssem):
    c = lax.axis_index("c")
    s = lax.axis_index("s")
    w = c * NS + s
    sl = pl.ds(s * ROWS_T, ROWS_T)
    pltpu.sync_copy(zeros_hbm.at[sl], acc.at[sl])
    plsc.subcore_barrier()

    _agg_pipeline(edge_hbm, w * E_W, lambda idx: rows_hbm.at[idx],
                  si, di, ti_v, rows_v, tail_v, acc, isem, gsem, ssem)
    plsc.subcore_barrier()
    pltpu.sync_copy(acc.at[sl], out_hbm.at[c, sl])


@functools.partial(
    pl.kernel,
    out_type=jax.ShapeDtypeStruct((N_PAD, HID), jnp.float32),
    mesh=_mesh,
    scratch_types=[
        pltpu.VMEM((ROWS_W, HID), jnp.float32),
        pltpu.VMEM((ROWS_W, HID), jnp.float32),
        pltpu.VMEM((ROWS_W,), jnp.float32),
        pltpu.VMEM((ROWS_W,), jnp.float32),
        pltpu.VMEM((ROWS_W,), jnp.float32),
        pltpu.VMEM((ROWS_W,), jnp.float32),
        pltpu.VMEM((ROWS_W,), jnp.float32),
        pltpu.VMEM((ROWS_W,), jnp.float32),
        pltpu.VMEM((HID,), jnp.float32),
    ],
    compiler_params=_sc_params,
)
def _sc_fin16(aggp_hbm, degin_hbm, degout_hbm, b1_hbm, h_hbm,
              p0_v, p1_v, di0_v, di1_v, do0_v, do1_v, nd_v, ns_v, b_v):
    c = lax.axis_index("c")
    s = lax.axis_index("s")
    w = c * NS + s
    slw = pl.ds(w * ROWS_W, ROWS_W)
    pltpu.sync_copy(aggp_hbm.at[0, slw], p0_v)
    pltpu.sync_copy(aggp_hbm.at[1, slw], p1_v)
    pltpu.sync_copy(degin_hbm.at[0, slw], di0_v)
    pltpu.sync_copy(degin_hbm.at[1, slw], di1_v)
    pltpu.sync_copy(degout_hbm.at[0, slw], do0_v)
    pltpu.sync_copy(degout_hbm.at[1, slw], do1_v)
    pltpu.sync_copy(b1_hbm, b_v)

    def nbody(j, carry):
        ds = pl.ds(j * 16, 16)
        nd_v[ds] = _rsqrt16(di0_v[ds] + di1_v[ds])
        ns_v[ds] = _rsqrt16(do0_v[ds] + do1_v[ds])
        return carry

    lax.fori_loop(0, ROWS_W // 16, nbody, 0)

    def rbody(j, carry):
        ndv = nd_v[pl.ds(j * 16, 16)]
        nsv = ns_v[pl.ds(j * 16, 16)]
        for t in range(16):
            i = j * 16 + t
            row = (p0_v[i, :] + p1_v[i, :]) * ndv[t] + b_v[:]
            p0_v[i, :] = jnp.maximum(row, 0.0) * nsv[t]
        return carry

    lax.fori_loop(0, ROWS_W // 16, rbody, 0)
    pltpu.sync_copy(p0_v, h_hbm.at[slw])


@functools.partial(
    pl.kernel,
    out_type=jax.ShapeDtypeStruct((N_PAD, C_PAD), jnp.float32),
    mesh=_mesh,
    scratch_types=[
        pltpu.VMEM((ROWS_W, C_PAD), jnp.float32),
        pltpu.VMEM((ROWS_W, C_PAD), jnp.float32),
        pltpu.VMEM((ROWS_W,), jnp.float32),
        pltpu.VMEM((ROWS_W,), jnp.float32),
        pltpu.VMEM((ROWS_W,), jnp.float32),
        pltpu.VMEM((C_PAD,), jnp.float32),
    ],
    compiler_params=_sc_params,
)
def _sc_fin48(aggp_hbm, degin_hbm, b2_hbm, out_hbm,
              p0_v, p1_v, di0_v, di1_v, nd_v, b_v):
    c = lax.axis_index("c")
    s = lax.axis_index("s")
    w = c * NS + s
    slw = pl.ds(w * ROWS_W, ROWS_W)
    pltpu.sync_copy(aggp_hbm.at[0, slw], p0_v)
    pltpu.sync_copy(aggp_hbm.at[1, slw], p1_v)
    pltpu.sync_copy(degin_hbm.at[0, slw], di0_v)
    pltpu.sync_copy(degin_hbm.at[1, slw], di1_v)
    pltpu.sync_copy(b2_hbm, b_v)

    def nbody(j, carry):
        ds = pl.ds(j * 16, 16)
        nd_v[ds] = _rsqrt16(di0_v[ds] + di1_v[ds])
        return carry

    lax.fori_loop(0, ROWS_W // 16, nbody, 0)

    def rbody(j, carry):
        ndv = nd_v[pl.ds(j * 16, 16)]
        for t in range(16):
            i = j * 16 + t
            for k in range(C_PAD // 16):
                ds = pl.ds(k * 16, 16)
                p0_v[i, ds] = (p0_v[i, ds] + p1_v[i, ds]) * ndv[t] + b_v[ds]
        return carry

    lax.fori_loop(0, ROWS_W // 16, rbody, 0)
    pltpu.sync_copy(p0_v, out_hbm.at[slw])


_B = 1024


def _tc_mm_body(x_ref, w_ref, o_ref):
    o_ref[...] = jnp.dot(x_ref[...], w_ref[...],
                         preferred_element_type=jnp.float32)


def _tc_mm(x, w, m_out):
    m, k = x.shape
    n = w.shape[1]
    return pl.pallas_call(
        _tc_mm_body,
        grid=(m_out // _B,),
        in_specs=[
            pl.BlockSpec((_B, k), lambda i: (i, 0)),
            pl.BlockSpec((k, n), lambda i: (0, 0)),
        ],
        out_specs=pl.BlockSpec((_B, n), lambda i: (i, 0)),
        out_shape=jax.ShapeDtypeStruct((m_out, n), jnp.float32),
    )(x, w)


def kernel(features, edge_index, W1, b1, W2, b2):
    w2p = jnp.pad(W2, ((0, 0), (0, C_PAD - N_CLASSES)))
    b2p = jnp.pad(b2, (0, C_PAD - N_CLASSES))

    ones_ch = jnp.ones((CH,), jnp.float32)
    zeros_1d = jnp.zeros((N_PAD,), jnp.float32)
    zeros_16 = jnp.zeros((N_PAD, HID), jnp.float32)
    zeros_48 = jnp.zeros((N_PAD, C_PAD), jnp.float32)

    degout_p, degin_p = _sc_degrees(edge_index, ones_ch, zeros_1d)
    xw_un = _tc_mm(features, W1, N_PAD)
    agg1_p = _sc_agg16(xw_un, degout_p, edge_index, zeros_16)
    h = _sc_fin16(agg1_p, degin_p, degout_p, b1)
    hw = _tc_mm(h, w2p, N_PAD)
    agg2_p = _sc_agg48(hw, edge_index, zeros_48)
    out = _sc_fin48(agg2_p, degin_p, b2p)
    return out[:N_NODES, :N_CLASSES]


# speedup vs baseline: 1.1035x; 1.1035x over previous
import functools

import jax
import jax.numpy as jnp
from jax import lax
from jax.experimental import pallas as pl
from jax.experimental.pallas import tpu as pltpu
from jax.experimental.pallas import tpu_sc as plsc

N_NODES = 10000
N_EDGES = 320000
F_IN = 128
HID = 16
N_CLASSES = 40

NC, NS = 2, 16
NW = NC * NS
N_PAD = 10240
ROWS_T = N_PAD // NS
ROWS_W = N_PAD // NW
E_W = N_EDGES // NW
CH = 120
NCH = E_W // CH
TAIL = E_W - NCH * CH
C_PAD = 48

_mesh = plsc.VectorSubcoreMesh(core_axis_name="c", subcore_axis_name="s",
                               num_cores=NC, num_subcores=NS)
_sc_params = pltpu.CompilerParams(use_tc_tiling_on_sc=False)


def _rsqrt16(d):
    x = jnp.maximum(d, 1.0)
    i = lax.bitcast_convert_type(x, jnp.int32)
    i = jnp.int32(0x5F3759DF) - lax.shift_right_arithmetic(i, 1)
    y = lax.bitcast_convert_type(i, jnp.float32)
    for _ in range(4):
        y = y * (1.5 - 0.5 * x * y * y)
    return y


_KI = 16
_NBUF, _DG, _DS = 12, 6, 6
_LEAD = 4
_HLEAD = 8


def _idx_ring(edge_hbm, ebase, si, di, isem):
    def start_l(i):
        off = ebase + i * CH
        pltpu.async_copy(edge_hbm.at[0, pl.ds(off, CH)],
                         si.at[lax.rem(i, _KI)], isem)
        pltpu.async_copy(edge_hbm.at[1, pl.ds(off, CH)],
                         di.at[lax.rem(i, _KI)], isem)

    def wait_l():
        pltpu.make_async_copy(edge_hbm.at[0, pl.ds(0, CH)],
                              si.at[0], isem).wait()
        pltpu.make_async_copy(edge_hbm.at[0, pl.ds(0, CH)],
                              di.at[0], isem).wait()

    return start_l, wait_l


def _agg_pipeline(edge_hbm, ebase, gather_src_at,
                  si, di, ti_v, rows_v, tail_v, acc, isem, gsem, ssem):
    start_l, wait_l = _idx_ring(edge_hbm, ebase, si, di, isem)

    def start_g(i):
        pltpu.async_copy(gather_src_at(si.at[lax.rem(i, _KI)]),
                         rows_v.at[lax.rem(i, _NBUF)], gsem)

    def wait_g():
        pltpu.make_async_copy(gather_src_at(si.at[0]),
                              rows_v.at[0], gsem).wait()

    def start_s(i):
        pltpu.async_copy(rows_v.at[lax.rem(i, _NBUF)],
                         acc.at[di.at[lax.rem(i, _KI)]], ssem, add=True)

    def wait_s():
        pltpu.make_async_copy(rows_v.at[0],
                              acc.at[di.at[0]], ssem).wait()

    for i in range(_DG + _LEAD):
        start_l(i)
    for i in range(_DG):
        wait_l()
        start_g(i)

    def body(i, carry):
        wait_g()
        pl.when(i >= _DS)(wait_s)
        pl.when(i + _DG + _LEAD < NCH)(lambda: start_l(i + _DG + _LEAD))

        def advance():
            wait_l()
            start_g(i + _DG)

        pl.when(i + _DG < NCH)(advance)
        start_s(i)
        return carry

    lax.fori_loop(0, NCH, body, 0)
    for _ in range(_DS):
        wait_s()

    toff = ebase + NCH * CH
    pltpu.sync_copy(edge_hbm.at[0, pl.ds(toff, TAIL)], ti_v.at[0])
    pltpu.sync_copy(edge_hbm.at[1, pl.ds(toff, TAIL)], ti_v.at[1])
    pltpu.async_copy(gather_src_at(ti_v.at[0]), tail_v, gsem).wait()
    pltpu.sync_copy(tail_v, acc.at[ti_v.at[1]], add=True)


@functools.partial(
    pl.kernel,
    out_type=(
        jax.ShapeDtypeStruct((NC, N_PAD), jnp.float32),
        jax.ShapeDtypeStruct((NC, N_PAD), jnp.float32),
    ),
    mesh=_mesh,
    scratch_types=[
        pltpu.VMEM((_KI, CH), jnp.int32),
        pltpu.VMEM((_KI, CH), jnp.int32),
        pltpu.VMEM((2, TAIL), jnp.int32),
        pltpu.VMEM((CH,), jnp.float32),
        pltpu.VMEM((TAIL,), jnp.float32),
        pltpu.VMEM_SHARED((N_PAD,), jnp.float32),
        pltpu.VMEM_SHARED((N_PAD,), jnp.float32),
        pltpu.SemaphoreType.DMA,
        pltpu.SemaphoreType.DMA,
    ],
    compiler_params=_sc_params,
)
def _sc_degrees(edge_hbm, ones_hbm, zeros_hbm,
                degout_hbm, degin_hbm,
                si, di, ti_v, ones_v, ones_t, acc_o, acc_i, isem, ssem):
    c = lax.axis_index("c")
    s = lax.axis_index("s")
    w = c * NS + s
    ebase = w * E_W
    sl = pl.ds(s * ROWS_T, ROWS_T)
    pltpu.sync_copy(zeros_hbm.at[sl], acc_o.at[sl])
    pltpu.sync_copy(zeros_hbm.at[sl], acc_i.at[sl])
    pltpu.sync_copy(ones_hbm, ones_v)
    pltpu.sync_copy(ones_hbm.at[pl.ds(0, TAIL)], ones_t)
    plsc.subcore_barrier()

    start_l, wait_l = _idx_ring(edge_hbm, ebase, si, di, isem)
    depth = 8

    def wait_s():
        pltpu.make_async_copy(ones_v, acc_o.at[si.at[0]], ssem).wait()

    def wait_s2():
        wait_s()
        wait_s()

    for i in range(_HLEAD):
        start_l(i)

    def body(i, carry):
        pl.when(i + _HLEAD < NCH)(lambda: start_l(i + _HLEAD))
        wait_l()
        pl.when(i >= depth)(wait_s2)
        m = lax.rem(i, _KI)
        pltpu.async_copy(ones_v, acc_o.at[si.at[m]], ssem, add=True)
        pltpu.async_copy(ones_v, acc_i.at[di.at[m]], ssem, add=True)
        return carry

    lax.fori_loop(0, NCH, body, 0)
    for _ in range(2 * depth):
        wait_s()

    toff = ebase + NCH * CH
    pltpu.sync_copy(edge_hbm.at[0, pl.ds(toff, TAIL)], ti_v.at[0])
    pltpu.sync_copy(edge_hbm.at[1, pl.ds(toff, TAIL)], ti_v.at[1])
    pltpu.sync_copy(ones_t, acc_o.at[ti_v.at[0]], add=True)
    pltpu.sync_copy(ones_t, acc_i.at[ti_v.at[1]], add=True)

    plsc.subcore_barrier()
    pltpu.sync_copy(acc_o.at[sl], degout_hbm.at[c, sl])
    pltpu.sync_copy(acc_i.at[sl], degin_hbm.at[c, sl])


@functools.partial(
    pl.kernel,
    out_type=jax.ShapeDtypeStruct((NC, N_PAD, HID), jnp.float32),
    mesh=_mesh,
    scratch_types=[
        pltpu.VMEM((_KI, CH), jnp.int32),
        pltpu.VMEM((_KI, CH), jnp.int32),
        pltpu.VMEM((2, TAIL), jnp.int32),
        pltpu.VMEM((_NBUF, CH, HID), jnp.float32),
        pltpu.VMEM((TAIL, HID), jnp.float32),
        pltpu.VMEM((ROWS_T, HID), jnp.float32),
        pltpu.VMEM((ROWS_T,), jnp.float32),
        pltpu.VMEM((ROWS_T,), jnp.float32),
        pltpu.VMEM((ROWS_T,), jnp.float32),
        pltpu.VMEM_SHARED((N_PAD, HID), jnp.float32),
        pltpu.VMEM_SHARED((N_PAD, HID), jnp.float32),
        pltpu.SemaphoreType.DMA,
        pltpu.SemaphoreType.DMA,
        pltpu.SemaphoreType.DMA,
    ],
    compiler_params=_sc_params,
)
def _sc_agg16(xwun_hbm, degs_hbm, edge_hbm, zeros_hbm, out_hbm,
              si, di, ti_v, rows_v, tail_v, xrows_v, dg0_v, dg1_v, norm_v,
              table, acc, isem, gsem, ssem):
    c = lax.axis_index("c")
    s = lax.axis_index("s")
    w = c * NS + s
    sl = pl.ds(s * ROWS_T, ROWS_T)

    pltpu.sync_copy(xwun_hbm.at[sl], xrows_v)
    pltpu.sync_copy(degs_hbm.at[0, sl], dg0_v)
    pltpu.sync_copy(degs_hbm.at[1, sl], dg1_v)
    pltpu.sync_copy(zeros_hbm.at[sl], acc.at[sl])

    def nbody(j, carry):
        d = dg0_v[pl.ds(j * 16, 16)] + dg1_v[pl.ds(j * 16, 16)]
        norm_v[pl.ds(j * 16, 16)] = _rsqrt16(d)
        return carry

    lax.fori_loop(0, ROWS_T // 16, nbody, 0)

    def sbody(j, carry):
        nv = norm_v[pl.ds(j * 16, 16)]
        for t in range(16):
            i = j * 16 + t
            xrows_v[i, :] = xrows_v[i, :] * nv[t]
        return carry

    lax.fori_loop(0, ROWS_T // 16, sbody, 0)
    pltpu.sync_copy(xrows_v, table.at[sl])
    plsc.subcore_barrier()

    _agg_pipeline(edge_hbm, w * E_W, lambda idx: table.at[idx],
                  si, di, ti_v, rows_v, tail_v, acc, isem, gsem, ssem)
    plsc.subcore_barrier()
    pltpu.sync_copy(acc.at[sl], out_hbm.at[c, sl])


@functools.partial(
    pl.kernel,
    out_type=jax.ShapeDtypeStruct((NC, N_PAD, C_PAD), jnp.float32),
    mesh=_mesh,
    scratch_types=[
        pltpu.VMEM((_KI, CH), jnp.int32),
        pltpu.VMEM((_KI, CH), jnp.int32),
        pltpu.VMEM((2, TAIL), jnp.int32),
        pltpu.VMEM((_NBUF, CH, C_PAD), jnp.float32),
        pltpu.VMEM((TAIL, C_PAD), jnp.float32),
        pltpu.VMEM_SHARED((N_PAD, C_PAD), jnp.float32),
        pltpu.SemaphoreType.DMA,
        pltpu.SemaphoreType.DMA,
        pltpu.SemaphoreType.DMA,
    ],
    compiler_params=_sc_params,
)
def _sc_agg48(rows_hbm, edge_hbm, zeros_hbm, out_hbm,
              si, di, ti_v, rows_v, tail_v, acc, isem, gsem, ssem):
    c = lax.axis_index("c")
    s = lax.axis_index("s")
    w = c * NS + s
    sl = pl.ds(s * ROWS_T, ROWS_T)
    pltpu.sync_copy(zeros_hbm.at[sl], acc.at[sl])
    plsc.subcore_barrier()

    _agg_pipeline(edge_hbm, w * E_W, lambda idx: rows_hbm.at[idx],
                  si, di, ti_v, rows_v, tail_v, acc, isem, gsem, ssem)
    plsc.subcore_barrier()
    pltpu.sync_copy(acc.at[sl], out_hbm.at[c, sl])


@functools.partial(
    pl.kernel,
    out_type=jax.ShapeDtypeStruct((N_PAD, HID), jnp.float32),
    mesh=_mesh,
    scratch_types=[
        pltpu.VMEM((ROWS_W, HID), jnp.float32),
        pltpu.VMEM((ROWS_W, HID), jnp.float32),
        pltpu.VMEM((ROWS_W,), jnp.float32),
        pltpu.VMEM((ROWS_W,), jnp.float32),
        pltpu.VMEM((ROWS_W,), jnp.float32),
        pltpu.VMEM((ROWS_W,), jnp.float32),
        pltpu.VMEM((ROWS_W,), jnp.float32),
        pltpu.VMEM((ROWS_W,), jnp.float32),
        pltpu.VMEM((HID,), jnp.float32),
    ],
    compiler_params=_sc_params,
)
def _sc_fin16(aggp_hbm, degin_hbm, degout_hbm, b1_hbm, h_hbm,
              p0_v, p1_v, di0_v, di1_v, do0_v, do1_v, nd_v, ns_v, b_v):
    c = lax.axis_index("c")
    s = lax.axis_index("s")
    w = c * NS + s
    slw = pl.ds(w * ROWS_W, ROWS_W)
    pltpu.sync_copy(aggp_hbm.at[0, slw], p0_v)
    pltpu.sync_copy(aggp_hbm.at[1, slw], p1_v)
    pltpu.sync_copy(degin_hbm.at[0, slw], di0_v)
    pltpu.sync_copy(degin_hbm.at[1, slw], di1_v)
    pltpu.sync_copy(degout_hbm.at[0, slw], do0_v)
    pltpu.sync_copy(degout_hbm.at[1, slw], do1_v)
    pltpu.sync_copy(b1_hbm, b_v)

    def nbody(j, carry):
        ds = pl.ds(j * 16, 16)
        nd_v[ds] = _rsqrt16(di0_v[ds] + di1_v[ds])
        ns_v[ds] = _rsqrt16(do0_v[ds] + do1_v[ds])
        return carry

    lax.fori_loop(0, ROWS_W // 16, nbody, 0)

    def rbody(j, carry):
        ndv = nd_v[pl.ds(j * 16, 16)]
        nsv = ns_v[pl.ds(j * 16, 16)]
        for t in range(16):
            i = j * 16 + t
            row = (p0_v[i, :] + p1_v[i, :]) * ndv[t] + b_v[:]
            p0_v[i, :] = jnp.maximum(row, 0.0) * nsv[t]
        return carry

    lax.fori_loop(0, ROWS_W // 16, rbody, 0)
    pltpu.sync_copy(p0_v, h_hbm.at[slw])


@functools.partial(
    pl.kernel,
    out_type=jax.ShapeDtypeStruct((N_PAD, C_PAD), jnp.float32),
    mesh=_mesh,
    scratch_types=[
        pltpu.VMEM((ROWS_W, C_PAD), jnp.float32),
        pltpu.VMEM((ROWS_W, C_PAD), jnp.float32),
        pltpu.VMEM((ROWS_W,), jnp.float32),
        pltpu.VMEM((ROWS_W,), jnp.float32),
        pltpu.VMEM((ROWS_W,), jnp.float32),
        pltpu.VMEM((C_PAD,), jnp.float32),
    ],
    compiler_params=_sc_params,
)
def _sc_fin48(aggp_hbm, degin_hbm, b2_hbm, out_hbm,
              p0_v, p1_v, di0_v, di1_v, nd_v, b_v):
    c = lax.axis_index("c")
    s = lax.axis_index("s")
    w = c * NS + s
    slw = pl.ds(w * ROWS_W, ROWS_W)
    pltpu.sync_copy(aggp_hbm.at[0, slw], p0_v)
    pltpu.sync_copy(aggp_hbm.at[1, slw], p1_v)
    pltpu.sync_copy(degin_hbm.at[0, slw], di0_v)
    pltpu.sync_copy(degin_hbm.at[1, slw], di1_v)
    pltpu.sync_copy(b2_hbm, b_v)

    def nbody(j, carry):
        ds = pl.ds(j * 16, 16)
        nd_v[ds] = _rsqrt16(di0_v[ds] + di1_v[ds])
        return carry

    lax.fori_loop(0, ROWS_W // 16, nbody, 0)

    def rbody(j, carry):
        ndv = nd_v[pl.ds(j * 16, 16)]
        for t in range(16):
            i = j * 16 + t
            for k in range(C_PAD // 16):
                ds = pl.ds(k * 16, 16)
                p0_v[i, ds] = (p0_v[i, ds] + p1_v[i, ds]) * ndv[t] + b_v[ds]
        return carry

    lax.fori_loop(0, ROWS_W // 16, rbody, 0)
    pltpu.sync_copy(p0_v, out_hbm.at[slw])


_B = 1024


def _tc_mm_body(x_ref, w_ref, o_ref):
    o_ref[...] = jnp.dot(x_ref[...], w_ref[...],
                         preferred_element_type=jnp.float32)


def _tc_mm(x, w, m_out):
    m, k = x.shape
    n = w.shape[1]
    return pl.pallas_call(
        _tc_mm_body,
        grid=(m_out // _B,),
        in_specs=[
            pl.BlockSpec((_B, k), lambda i: (i, 0)),
            pl.BlockSpec((k, n), lambda i: (0, 0)),
        ],
        out_specs=pl.BlockSpec((_B, n), lambda i: (i, 0)),
        out_shape=jax.ShapeDtypeStruct((m_out, n), jnp.float32),
    )(x, w)


def kernel(features, edge_index, W1, b1, W2, b2):
    w2p = jnp.pad(W2, ((0, 0), (0, C_PAD - N_CLASSES)))
    b2p = jnp.pad(b2, (0, C_PAD - N_CLASSES))

    ones_ch = jnp.ones((CH,), jnp.float32)
    zeros_1d = jnp.zeros((N_PAD,), jnp.float32)
    zeros_16 = jnp.zeros((N_PAD, HID), jnp.float32)
    zeros_48 = jnp.zeros((N_PAD, C_PAD), jnp.float32)

    degout_p, degin_p = _sc_degrees(edge_index, ones_ch, zeros_1d)
    xw_un = _tc_mm(features, W1, N_PAD)
    agg1_p = _sc_agg16(xw_un, degout_p, edge_index, zeros_16)
    h = _sc_fin16(agg1_p, degin_p, degout_p, b1)
    hw = _tc_mm(h, w2p, N_PAD)
    agg2_p = _sc_agg48(hw, edge_index, zeros_48)
    out = _sc_fin48(agg2_p, degin_p, b2p)
    return out[:N_NODES, :N_CLASSES]
